# Initial kernel scaffold; baseline (speedup 1.0000x reference)
import jax
import jax.numpy as jnp
from jax.experimental import pallas as pl


def kernel(x, edge_index, W1, b1, W2, b2, att_w):
    raise NotImplementedError("write your pallas kernel here")


# speedup vs baseline: 12.2764x; 12.2764x over previous
import functools

import jax
import jax.numpy as jnp
from jax import lax
from jax.experimental import pallas as pl
from jax.experimental.pallas import tpu as pltpu
from jax.experimental.pallas import tpu_sc as plsc

NC = 2
NS = 16
NW = NC * NS
CH = 80
BR = 1000


def _sc_mesh():
    return plsc.VectorSubcoreMesh(
        core_axis_name="c", subcore_axis_name="s", num_cores=NC, num_subcores=NS
    )


def _pad_rows(n):
    return NS * ((n + 8 * NS - 1) // (8 * NS)) * 8 // NS


@functools.cache
def _make_deg_kernel(n, e):
    epw = e // NW
    nch = epw // CH
    r = _pad_rows(n)
    np_ = NS * r

    def body(dst_hbm, zeros_hbm, ones_hbm, out_hbm, deg_sh, idx_v, ones_v, sem):
        cid = lax.axis_index("c")
        sid = lax.axis_index("s")
        wid = cid * NS + sid
        pltpu.sync_copy(ones_hbm, ones_v)
        pltpu.sync_copy(zeros_hbm, deg_sh.at[pl.ds(sid * r, r)])
        plsc.subcore_barrier()

        def chunk(k, carry):
            off = wid * epw + k * CH
            pltpu.sync_copy(dst_hbm.at[pl.ds(off, CH)], idx_v)
            pltpu.sync_copy(ones_v, deg_sh.at[idx_v], add=True)
            return carry

        lax.fori_loop(0, nch, chunk, 0)
        plsc.subcore_barrier()
        pltpu.sync_copy(deg_sh.at[pl.ds(sid * r, r)],
                        out_hbm.at[cid, pl.ds(sid * r, r)])

    return pl.kernel(
        body,
        out_type=jax.ShapeDtypeStruct((NC, np_, 16), jnp.float32),
        mesh=_sc_mesh(),
        scratch_types=[
            pltpu.VMEM_SHARED((np_, 16), jnp.float32),
            pltpu.VMEM((CH,), jnp.int32),
            pltpu.VMEM((CH, 16), jnp.float32),
            pltpu.SemaphoreType.DMA,
        ],
    )


@functools.cache
def _make_scatter_kernel(n, e, h):
    epw = e // NW
    nch = epw // CH
    r = _pad_rows(n)
    np_ = NS * r
    tail = n - (NS - 1) * r
    ztail = np_ - n

    def body(g_hbm, src_hbm, dst_hbm, zeros_hbm, out_hbm,
             acc_sh, sidx_v, didx_v, rows_v, sem):
        cid = lax.axis_index("c")
        sid = lax.axis_index("s")
        wid = cid * NS + sid

        pltpu.sync_copy(zeros_hbm, acc_sh.at[pl.ds(sid * r, r)])
        plsc.subcore_barrier()

        def chunk(k, carry):
            off = wid * epw + k * CH
            pltpu.sync_copy(src_hbm.at[pl.ds(off, CH)], sidx_v)
            pltpu.sync_copy(dst_hbm.at[pl.ds(off, CH)], didx_v)
            pltpu.async_copy(g_hbm.at[sidx_v], rows_v, sem).wait()
            pltpu.sync_copy(rows_v, acc_sh.at[didx_v], add=True)
            return carry

        lax.fori_loop(0, nch, chunk, 0)
        plsc.subcore_barrier()
        pltpu.sync_copy(acc_sh.at[pl.ds(sid * r, r)],
                        out_hbm.at[cid, pl.ds(sid * r, r)])

    return pl.kernel(
        body,
        out_type=jax.ShapeDtypeStruct((NC, np_, h), jnp.float32),
        mesh=_sc_mesh(),
        compiler_params=pltpu.CompilerParams(use_tc_tiling_on_sc=True),
        scratch_types=[
            pltpu.VMEM_SHARED((np_, h), jnp.float32),
            pltpu.VMEM((CH,), jnp.int32),
            pltpu.VMEM((CH,), jnp.int32),
            pltpu.VMEM((CH, h), jnp.float32),
            pltpu.SemaphoreType.DMA,
        ],
    )


def _row_spec(w):
    return pl.BlockSpec((BR, w), lambda i: (i, 0))


def _full_spec(a, b):
    return pl.BlockSpec((a, b), lambda i: (0, 0))


def _tc_dinv(degp, n):
    def body(deg_ref, o_ref):
        deg = deg_ref[0, :, 0:1] + deg_ref[1, :, 0:1] + 1.0
        o_ref[...] = jnp.broadcast_to(lax.rsqrt(deg), o_ref.shape)

    return pl.pallas_call(
        body,
        grid=(n // BR,),
        in_specs=[pl.BlockSpec((NC, BR, 16), lambda i: (0, i, 0))],
        out_specs=_row_spec(128),
        out_shape=jax.ShapeDtypeStruct((n, 128), jnp.float32),
    )(degp)


def _tc_first(x, w1, dinv):
    n, d = x.shape
    h = w1.shape[1]

    def body(x_ref, w_ref, dinv_ref, g_ref):
        g_ref[...] = dinv_ref[...] * jnp.dot(
            x_ref[...], w_ref[...], preferred_element_type=jnp.float32)

    return pl.pallas_call(
        body,
        grid=(n // BR,),
        in_specs=[_row_spec(d), _full_spec(d, h), _row_spec(128)],
        out_specs=_row_spec(h),
        out_shape=jax.ShapeDtypeStruct((n, h), jnp.float32),
    )(x, w1, dinv)


def _tc_accsum(accp, n, h):
    def body(acc_ref, o_ref):
        o_ref[...] = acc_ref[0] + acc_ref[1]

    return pl.pallas_call(
        body,
        grid=(n // BR,),
        in_specs=[pl.BlockSpec((NC, BR, h), lambda i: (0, i, 0))],
        out_specs=_row_spec(h),
        out_shape=jax.ShapeDtypeStruct((n, h), jnp.float32),
    )(accp)


def _tc_mid(acc, g1, dinv, b1, w2):
    n, h = g1.shape

    def body(acc_ref, g_ref, dinv_ref, b_ref, w_ref, o_ref):
        dv = dinv_ref[...]
        h1 = jnp.maximum(dv * (acc_ref[...] + g_ref[...]) + b_ref[...], 0.0)
        o_ref[...] = dv * jnp.dot(h1, w_ref[...],
                                  preferred_element_type=jnp.float32)

    return pl.pallas_call(
        body,
        grid=(n // BR,),
        in_specs=[
            _row_spec(h), _row_spec(h), _row_spec(128),
            _full_spec(1, h), _full_spec(h, h),
        ],
        out_specs=_row_spec(h),
        out_shape=jax.ShapeDtypeStruct((n, h), jnp.float32),
    )(acc, g1, dinv, b1, w2)


def _tc_h2(acc, g2, dinv, b2):
    n, h = g2.shape

    def body(acc_ref, g_ref, dinv_ref, b_ref, o_ref):
        pre = dinv_ref[...] * (acc_ref[...] + g_ref[...]) + b_ref[...]
        o_ref[...] = jnp.maximum(pre, 0.0)

    return pl.pallas_call(
        body,
        grid=(n // BR,),
        in_specs=[
            _row_spec(h), _row_spec(h), _row_spec(128), _full_spec(1, h),
        ],
        out_specs=_row_spec(h),
        out_shape=jax.ShapeDtypeStruct((n, h), jnp.float32),
    )(acc, g2, dinv, b2)


def _tc_scores(h2, att_w):
    n, h = h2.shape

    def body(h_ref, aw_ref, s_ref):
        s_ref[...] = jnp.dot(h_ref[...], aw_ref[...],
                             preferred_element_type=jnp.float32)

    return pl.pallas_call(
        body,
        grid=(n // BR,),
        in_specs=[_row_spec(h), _full_spec(h, 1)],
        out_specs=_row_spec(1),
        out_shape=jax.ShapeDtypeStruct((n, 1), jnp.float32),
    )(h2, att_w)


def _tc_probs(scores):
    n = scores.shape[0]

    def body(s_ref, p_ref):
        s = s_ref[...]
        m = jnp.max(s)
        e = jnp.exp(s - m)
        p_ref[...] = e / jnp.sum(e)

    return pl.pallas_call(
        body,
        in_specs=[pl.BlockSpec((n, 1), lambda: (0, 0))],
        out_specs=pl.BlockSpec((n, 1), lambda: (0, 0)),
        out_shape=jax.ShapeDtypeStruct((n, 1), jnp.float32),
    )(scores)


def _tc_attend(h2, probs):
    n, h = h2.shape

    def body(h_ref, p_ref, o_ref):
        o_ref[...] = h_ref[...] * p_ref[...]

    return pl.pallas_call(
        body,
        grid=(n // BR,),
        in_specs=[_row_spec(h), _row_spec(1)],
        out_specs=_row_spec(h),
        out_shape=jax.ShapeDtypeStruct((n, h), jnp.float32),
    )(h2, probs)


def kernel(x, edge_index, W1, b1, W2, b2, att_w):
    n, d = x.shape
    h = W1.shape[1]
    e = edge_index.shape[1]
    src = edge_index[0]
    dst = edge_index[1]
    r = _pad_rows(n)

    zeros16 = jnp.zeros((r, 16), jnp.float32)
    ones16 = jnp.ones((CH, 16), jnp.float32)
    zrows = jnp.zeros((r, h), jnp.float32)
    b1r = b1.reshape(1, h)
    b2r = b2.reshape(1, h)

    degp = _make_deg_kernel(n, e)(dst, zeros16, ones16)
    dinv = _tc_dinv(degp, n)
    g1 = _tc_first(x, W1, dinv)
    acc1p = _make_scatter_kernel(n, e, h)(g1, src, dst, zrows)
    acc1 = _tc_accsum(acc1p, n, h)
    g2 = _tc_mid(acc1, g1, dinv, b1r, W2)
    acc2p = _make_scatter_kernel(n, e, h)(g2, src, dst, zrows)
    acc2 = _tc_accsum(acc2p, n, h)
    h2 = _tc_h2(acc2, g2, dinv, b2r)
    scores = _tc_scores(h2, att_w)
    probs = _tc_probs(scores)
    attended = _tc_attend(h2, probs)
    return (attended, probs.reshape(n))
